# Initial kernel scaffold; baseline (speedup 1.0000x reference)
import jax
import jax.numpy as jnp
from jax.experimental import pallas as pl


def kernel(x, router_w, router_b, ln_g, ln_b, gate_w, gate_b):
    raise NotImplementedError("write your pallas kernel here")


# speedup vs baseline: 1.7903x; 1.7903x over previous
import functools

import jax
import jax.numpy as jnp
from jax import lax
from jax.experimental import pallas as pl
from jax.experimental.pallas import tpu as pltpu

_TOPK_FRAC = 0.12
_LN_EPS = 1e-5
_BS_BLOCK = 4


def _sortable_int(v):
    i = lax.bitcast_convert_type(v, jnp.int32)
    return jnp.where(i < 0, i ^ jnp.int32(0x7FFFFFFF), i)


def _topk_mask(r, k):
    b, s = r.shape
    key = _sortable_int(r)
    kk = jnp.int32(k)

    def val_step(_, lohi):
        lo, hi = lohi
        diff = hi - lo
        mid = lo + (lax.shift_right_logical(diff, 1) + (diff & 1))
        cnt = jnp.sum((key >= mid).astype(jnp.int32), axis=-1, keepdims=True)
        ok = cnt >= kk
        return jnp.where(ok, mid, lo), jnp.where(ok, hi, mid - 1)

    lo0 = jnp.full((b, 1), jnp.iinfo(jnp.int32).min, jnp.int32)
    hi0 = jnp.full((b, 1), jnp.iinfo(jnp.int32).max, jnp.int32)
    t, _ = lax.fori_loop(0, 32, val_step, (lo0, hi0))

    gt = key > t
    eq = key == t
    need = kk - jnp.sum(gt.astype(jnp.int32), axis=-1, keepdims=True)
    idx = lax.broadcasted_iota(jnp.int32, (b, s), 1)

    def idx_step(_, lohi):
        lo, hi = lohi
        mid = lax.shift_right_logical(lo + hi, 1)
        cnt = jnp.sum((eq & (idx <= mid)).astype(jnp.int32), axis=-1,
                      keepdims=True)
        ok = cnt >= need
        return jnp.where(ok, lo, mid + 1), jnp.where(ok, mid, hi)

    lo0 = jnp.zeros((b, 1), jnp.int32)
    hi0 = jnp.full((b, 1), s - 1, jnp.int32)
    cut, _ = lax.fori_loop(0, max(1, (s - 1).bit_length()), idx_step,
                           (lo0, hi0))
    return gt | (eq & (idx <= cut))


def _fused_block(x_ref, rw_ref, rb_ref, g_ref, b_ref, gw_ref, gb_ref,
                 out_ref, *, k):
    x = x_ref[...]
    bb, s, d = x.shape
    xb = x.astype(jnp.bfloat16).astype(jnp.float32)
    rwb = rw_ref[...].astype(jnp.bfloat16).astype(jnp.float32)
    r = jnp.sum(xb * rwb, axis=-1) + rb_ref[0, 0]
    w = jnp.where(_topk_mask(r, k), r, jnp.float32(0.0))
    mu = jnp.mean(x, axis=-1, keepdims=True)
    xc = x - mu
    var = jnp.mean(xc * xc, axis=-1, keepdims=True)
    ln = xc / jnp.sqrt(var + _LN_EPS) * g_ref[...] + b_ref[...]
    y = x + ln * w[..., None]
    o = lax.dot_general(y.reshape(bb * s, d), gw_ref[...],
                        (((1,), (1,)), ((), ())),
                        preferred_element_type=jnp.float32)
    out_ref[...] = o.reshape(bb, s, d) + gb_ref[...]


def kernel(x, router_w, router_b, ln_g, ln_b, gate_w, gate_b):
    bs, s, d = x.shape
    k = int(_TOPK_FRAC * s)
    blk = _BS_BLOCK
    while bs % blk:
        blk //= 2
    rw = router_w.reshape(1, 1, d).astype(jnp.float32)
    rb = jnp.asarray(router_b, jnp.float32).reshape(1, 1)
    g = ln_g.reshape(1, 1, d).astype(jnp.float32)
    b = ln_b.reshape(1, 1, d).astype(jnp.float32)
    gb = gate_b.reshape(1, 1, d).astype(jnp.float32)
    return pl.pallas_call(
        functools.partial(_fused_block, k=k),
        grid=(bs // blk,),
        in_specs=[
            pl.BlockSpec((blk, s, d), lambda i: (i, 0, 0)),
            pl.BlockSpec((1, 1, d), lambda i: (0, 0, 0)),
            pl.BlockSpec((1, 1), lambda i: (0, 0)),
            pl.BlockSpec((1, 1, d), lambda i: (0, 0, 0)),
            pl.BlockSpec((1, 1, d), lambda i: (0, 0, 0)),
            pl.BlockSpec((d, d), lambda i: (0, 0)),
            pl.BlockSpec((1, 1, d), lambda i: (0, 0, 0)),
        ],
        out_specs=pl.BlockSpec((blk, s, d), lambda i: (i, 0, 0)),
        out_shape=jax.ShapeDtypeStruct((bs, s, d), jnp.float32),
        compiler_params=pltpu.CompilerParams(
            dimension_semantics=("parallel",)),
    )(x, rw, rb, g, b, gate_w, gb)
